# Initial kernel scaffold; baseline (speedup 1.0000x reference)
import jax
import jax.numpy as jnp
from jax.experimental import pallas as pl


def kernel(x, edge_index_t, adj_old, edge_importance, W1, b1, W2, b2, W3, b3, Wt, bt, gamma, beta):
    raise NotImplementedError("write your pallas kernel here")


# speedup vs baseline: 6.9062x; 6.9062x over previous
import functools

import jax
import jax.numpy as jnp
from jax import lax
from jax.experimental import pallas as pl
from jax.experimental.pallas import tpu as pltpu
from jax.experimental.pallas import tpu_sc as plsc

T = 4096
N = 25
C = 64
F = N * C
E = 16384
NCHUNK = 8
CHK = F // NCHUNK
TB = 512
NTILE = 16
EB = 128
EDGES_PER_TILE = E // NTILE
ROWS_PER_TILE = T // NTILE


def _dense_body(x_ref, q_ref, w_ref, bias_ref, gamma_ref, beta_ref, hw_ref):
    xb = x_ref[...]
    h = jnp.dot(xb, q_ref[...], preferred_element_type=jnp.float32)
    h = h + bias_ref[...]
    mu = jnp.mean(h, axis=-1, keepdims=True)
    hc = h - mu
    var = jnp.mean(hc * hc, axis=-1, keepdims=True)
    hn = hc * lax.rsqrt(var + 1e-5)
    hn = hn * gamma_ref[...] + beta_ref[...]
    hr = jnp.maximum(hn, 0.0)
    for c in range(NCHUNK):
        hw_ref[c] = jnp.dot(hr, w_ref[c], preferred_element_type=jnp.float32)


def _epilogue_body(hw_ref, acc_ref, bt_ref, out_ref):
    parts = [acc_ref[c] + hw_ref[c] for c in range(NCHUNK)]
    cat = jnp.concatenate(parts, axis=-1)
    out_ref[...] = jnp.maximum(cat + bt_ref[...], 0.0)


def _make_edge_scatter():
    mesh = plsc.VectorSubcoreMesh(core_axis_name="c", subcore_axis_name="s")

    @functools.partial(
        pl.kernel,
        mesh=mesh,
        compiler_params=pltpu.CompilerParams(use_tc_tiling_on_sc=False),
        out_type=jax.ShapeDtypeStruct((NCHUNK, T, CHK), jnp.float32),
        scratch_types=[
            pltpu.VMEM((EB,), jnp.int32),
            pltpu.VMEM((EB,), jnp.int32),
            pltpu.VMEM((EB, CHK), jnp.float32),
            pltpu.VMEM((EB, CHK), jnp.float32),
            pltpu.VMEM_SHARED((T, CHK), jnp.float32),
            pltpu.SemaphoreType.DMA,
        ],
    )
    def edge_scatter(hw4_hbm, src_hbm, dst_hbm, zeros_hbm, acc4_hbm,
                     idx_s, idx_d, rows, zrows, acc_sh, sem):
        cid = lax.axis_index("c")
        sid = lax.axis_index("s")
        r0 = sid * ROWS_PER_TILE
        pltpu.sync_copy(zeros_hbm, zrows)

        def do_chunk(c):
            hw_c = hw4_hbm.at[c]
            acc_c = acc4_hbm.at[c]
            for rr in range(ROWS_PER_TILE // EB):
                pltpu.sync_copy(zrows, acc_sh.at[pl.ds(r0 + rr * EB, EB)])
            plsc.subcore_barrier()
            e0 = sid * EDGES_PER_TILE
            for b in range(EDGES_PER_TILE // EB):
                base = e0 + b * EB
                pltpu.sync_copy(src_hbm.at[pl.ds(base, EB)], idx_s)
                pltpu.sync_copy(dst_hbm.at[pl.ds(base, EB)], idx_d)
                pltpu.async_copy(hw_c.at[idx_s], rows, sem).wait()
                pltpu.sync_copy(rows, acc_sh.at[idx_d], add=True)
            plsc.subcore_barrier()
            for rr in range(ROWS_PER_TILE // EB):
                sl = pl.ds(r0 + rr * EB, EB)
                pltpu.sync_copy(acc_sh.at[sl], acc_c.at[sl])

        half = NCHUNK // 2

        @pl.when(cid == 0)
        def _():
            for c in range(half):
                do_chunk(c)

        @pl.when(cid == 1)
        def _():
            for c in range(half, NCHUNK):
                do_chunk(c)

    return edge_scatter


def _gcn_norm(A):
    A = A + jnp.eye(N, dtype=A.dtype)
    d = A.sum(axis=-1)
    dinv = jnp.where(d > 0, 1.0 / jnp.sqrt(d), 0.0)
    return dinv[:, None] * A * dinv[None, :]


def kernel(x, edge_index_t, adj_old, edge_importance,
           W1, b1, W2, b2, W3, b3, Wt, bt, gamma, beta):
    A = adj_old * edge_importance
    Q = jnp.zeros((F, F), jnp.float32)
    for Ai, Wi in ((A[0], W1), (A[1], W2), (A[2], W3)):
        An = _gcn_norm(Ai)
        Q = Q + jnp.einsum('nm,kc->mknc', An, Wi).reshape(F, F)
    bias_all = jnp.tile(b1 + b2 + b3, N)[None, :]
    WtBD = jnp.kron(jnp.eye(N, dtype=jnp.float32), Wt)
    wstack = WtBD.reshape(F, NCHUNK, CHK).transpose(1, 0, 2)
    bt_all = jnp.tile(bt, N)[None, :]

    xf = x.reshape(T, F)
    src = edge_index_t[0].astype(jnp.int32)
    dst = edge_index_t[1].astype(jnp.int32)
    zeros_eb = jnp.zeros((EB, CHK), jnp.float32)

    nblk = T // TB
    hw4 = pl.pallas_call(
        _dense_body,
        grid=(nblk,),
        in_specs=[
            pl.BlockSpec((TB, F), lambda i: (i, 0)),
            pl.BlockSpec((F, F), lambda i: (0, 0)),
            pl.BlockSpec((NCHUNK, F, CHK), lambda i: (0, 0, 0)),
            pl.BlockSpec((1, F), lambda i: (0, 0)),
            pl.BlockSpec((1, F), lambda i: (0, 0)),
            pl.BlockSpec((1, F), lambda i: (0, 0)),
        ],
        out_specs=pl.BlockSpec((NCHUNK, TB, CHK), lambda i: (0, i, 0)),
        out_shape=jax.ShapeDtypeStruct((NCHUNK, T, CHK), jnp.float32),
    )(xf, Q, wstack, bias_all, gamma[None, :], beta[None, :])

    acc4 = _make_edge_scatter()(hw4, src, dst, zeros_eb)

    out = pl.pallas_call(
        _epilogue_body,
        grid=(nblk,),
        in_specs=[
            pl.BlockSpec((NCHUNK, TB, CHK), lambda i: (0, i, 0)),
            pl.BlockSpec((NCHUNK, TB, CHK), lambda i: (0, i, 0)),
            pl.BlockSpec((1, F), lambda i: (0, 0)),
        ],
        out_specs=pl.BlockSpec((TB, F), lambda i: (i, 0)),
        out_shape=jax.ShapeDtypeStruct((T, F), jnp.float32),
    )(hw4, acc4, bt_all)

    return out.reshape(T, N, C)
